# baseline (speedup 1.0000x reference)
import dataclasses

import jax
import jax.numpy as jnp
import numpy as np
from jax import lax
from jax.experimental import pallas as pl
from jax.experimental.pallas import tpu as pltpu
from jax.experimental.pallas import tpu_sc as plsc

B, C, H, W = 16, 3, 224, 224
G = 256
MC = 8
P = H * W
HROWS = H // 2
HB = 16
NLANE = 16
NC, NS = 2, 16

_MESH = plsc.VectorSubcoreMesh(core_axis_name="core", subcore_axis_name="subcore")


def _fixed_logistic_noise():
    u = jax.random.uniform(jax.random.key(42), (B, G, MC),
                           minval=1e-6, maxval=1.0 - 1e-6)
    lnoise = jnp.log(u) - jnp.log1p(-u)
    return np.asarray(lnoise.transpose(0, 2, 1).reshape(B * MC * G))


_LFLAT = _fixed_logistic_noise()

_CP = pltpu.CompilerParams(use_tc_tiling_on_sc=True)
if "needs_layout_passes" in pltpu.CompilerParams.__dataclass_fields__:
    _CP = dataclasses.replace(_CP, needs_layout_passes=False)


def _worker_id():
    return lax.axis_index("core") * NS + lax.axis_index("subcore")


def _sigmoid(v):
    return 1.0 / (1.0 + jnp.exp(-v))


def _bf16_round(v):
    y = plsc.bitcast(v, jnp.uint32)
    r = (y + jnp.uint32(0x7FFF) + ((y >> jnp.uint32(16)) & jnp.uint32(1)))
    r = r & jnp.uint32(0xFFFF0000)
    return plsc.bitcast(r, jnp.float32)


NBANDS = HROWS // HB


def _k1_body(x_hbm, g_hbm, w_hbm, l_hbm, mask_hbm, gp_hbm, pp_hbm,
             xv0, xv1, gv0, gv1, pv0, pv1, sums, counts, pvs, pvc,
             lv, gpv, table, sv0, sv1, wv, shared,
             semx, semg, semp, semm):
    wid = _worker_id()
    sid = lax.axis_index("subcore")
    b = wid // 2
    half = wid % 2
    hbase = half * HROWS
    xvs, gvs, pvs2 = (xv0, xv1), (gv0, gv1), (pv0, pv1)

    cl = pltpu.async_copy(l_hbm.at[pl.ds(b * MC * G, MC * G)], lv, semm)
    pltpu.sync_copy(w_hbm, wv)
    w0 = wv[pl.ds(0, NLANE)]
    w1 = wv[pl.ds(NLANE, NLANE)]
    w2 = wv[pl.ds(2 * NLANE, NLANE)]
    bias = wv[pl.ds(3 * NLANE, NLANE)]

    zero = jnp.zeros((NLANE,), jnp.float32)
    ones = jnp.full((NLANE,), 1.0, jnp.float32)

    @pl.loop(0, G, step=NLANE)
    def _(g):
        sums[pl.ds(g, NLANE)] = zero
        counts[pl.ds(g, NLANE)] = zero

    def fetch_a(k):
        h0 = hbase + k * HB
        xc = pltpu.async_copy(x_hbm.at[b, :, pl.ds(h0, HB)], xvs[k % 2], semx)
        gc = pltpu.async_copy(g_hbm.at[b, pl.ds(h0, HB)], gvs[k % 2], semg)
        return xc, gc

    pend = fetch_a(0)
    ppcop = [None] * NBANDS
    for k in range(NBANDS):
        xc, gc = pend
        xc.wait()
        gc.wait()
        if k + 1 < NBANDS:
            pend = fetch_a(k + 1)
        if k >= 2:
            ppcop[k - 2].wait()
        xv, gv, ppv = xvs[k % 2], gvs[k % 2], pvs2[k % 2]

        @pl.loop(0, HB)
        def _(r):
            for w in range(0, W, NLANE):
                sl = (r, pl.ds(w, NLANE))
                x0r = _bf16_round(xv[0, r, pl.ds(w, NLANE)])
                x1r = _bf16_round(xv[1, r, pl.ds(w, NLANE)])
                x2r = _bf16_round(xv[2, r, pl.ds(w, NLANE)])
                e = x0r * w0 + x1r * w1 + x2r * w2 + bias
                ppv[sl] = _sigmoid(e)
                g = gv[sl]
                plsc.addupdate_scatter(sums, [g], e)
                plsc.addupdate_scatter(counts, [g], ones)

        h0 = hbase + k * HB
        ppcop[k] = pltpu.async_copy(ppv, pp_hbm.at[b, pl.ds(h0, HB)], semp)

    ppcop[NBANDS - 2].wait()
    ppcop[NBANDS - 1].wait()

    pltpu.sync_copy(sums, shared.at[sid, 0])
    pltpu.sync_copy(counts, shared.at[sid, 1])
    plsc.subcore_barrier()
    pltpu.sync_copy(shared.at[sid ^ 1, 0], pvs)
    pltpu.sync_copy(shared.at[sid ^ 1, 1], pvc)
    cl.wait()

    iota8 = lax.iota(jnp.int32, NLANE) * MC

    @pl.loop(0, G, step=NLANE)
    def _(g):
        sl = pl.ds(g, NLANE)
        s = sums[sl] + pvs[sl]
        n = counts[sl] + pvc[sl]
        mu = s / jnp.maximum(n, 1.0)
        gpv[sl] = _sigmoid(mu)
        for m in range(MC):
            hard = jnp.where(mu + lv[pl.ds(m * G + g, NLANE)] > 0.0, 1.0, 0.0)
            plsc.store_scatter(table, [iota8 + (g * MC + m)], hard)

    @pl.when(half == 0)
    def _():
        pltpu.sync_copy(gpv, gp_hbm.at[pl.ds(b * G, G)])

    svs = (sv0, sv1)

    def fetch_b(k):
        h0 = hbase + k * HB
        return pltpu.async_copy(g_hbm.at[b, pl.ds(h0, HB)], gvs[k % 2], semg)

    pend = fetch_b(0)
    mcop = [None] * NBANDS
    for k in range(NBANDS):
        pend.wait()
        if k + 1 < NBANDS:
            pend = fetch_b(k + 1)
        if k >= 2:
            for hc in mcop[k - 2]:
                hc.wait()
        gv, selv = gvs[k % 2], svs[k % 2]

        @pl.loop(0, HB)
        def _(r):
            for w in range(0, W, NLANE):
                g8 = gv[r, pl.ds(w, NLANE)] * MC
                for m in range(MC):
                    selv[r, m, pl.ds(w, NLANE)] = plsc.load_gather(
                        table, [g8 + m])

        h0 = hbase + k * HB
        mcop[k] = [
            pltpu.async_copy(selv, mask_hbm.at[b, c, pl.ds(h0, HB)], semm)
            for c in range(C)
        ]

    for k in (NBANDS - 2, NBANDS - 1):
        for hc in mcop[k]:
            hc.wait()


def kernel(x, groups, W_pred, b_pred):
    w0bf = W_pred[0].astype(jnp.bfloat16).astype(jnp.float32)
    wvec = jnp.concatenate([w0bf, b_pred[0:1]])
    wflat = jnp.broadcast_to(wvec[:, None], (4, NLANE)).reshape(4 * NLANE)

    lflat = jnp.asarray(_LFLAT)

    k1 = pl.kernel(
        _k1_body,
        out_type=[
            jax.ShapeDtypeStruct((B, C, H, MC, W), jnp.float32),
            jax.ShapeDtypeStruct((B * G,), jnp.float32),
            jax.ShapeDtypeStruct((B, H, W), jnp.float32),
        ],
        mesh=_MESH,
        compiler_params=_CP,
        scratch_types=[
            pltpu.VMEM((C, HB, W), jnp.float32),
            pltpu.VMEM((C, HB, W), jnp.float32),
            pltpu.VMEM((HB, W), jnp.int32),
            pltpu.VMEM((HB, W), jnp.int32),
            pltpu.VMEM((HB, W), jnp.float32),
            pltpu.VMEM((HB, W), jnp.float32),
            pltpu.VMEM((G,), jnp.float32),
            pltpu.VMEM((G,), jnp.float32),
            pltpu.VMEM((G,), jnp.float32),
            pltpu.VMEM((G,), jnp.float32),
            pltpu.VMEM((MC * G,), jnp.float32),
            pltpu.VMEM((G,), jnp.float32),
            pltpu.VMEM((G * MC,), jnp.float32),
            pltpu.VMEM((HB, MC, W), jnp.float32),
            pltpu.VMEM((HB, MC, W), jnp.float32),
            pltpu.VMEM((4 * NLANE,), jnp.float32),
            pltpu.VMEM_SHARED((NS, 2, G), jnp.float32),
            pltpu.SemaphoreType.DMA,
            pltpu.SemaphoreType.DMA,
            pltpu.SemaphoreType.DMA,
            pltpu.SemaphoreType.DMA,
        ],
    )
    maskT, group_probs, pp = k1(x, groups, wflat, lflat)

    mask = maskT.transpose(0, 1, 2, 4, 3)
    return (mask, group_probs.reshape(B, G), pp)


# speedup vs baseline: 1.0457x; 1.0457x over previous
import dataclasses

import jax
import jax.numpy as jnp
import numpy as np
from jax import lax
from jax.experimental import pallas as pl
from jax.experimental.pallas import tpu as pltpu
from jax.experimental.pallas import tpu_sc as plsc

B, C, H, W = 16, 3, 224, 224
G = 256
MC = 8
P = H * W
HROWS = H // 2
HB = 16
NLANE = 16
NC, NS = 2, 16

_MESH = plsc.VectorSubcoreMesh(core_axis_name="core", subcore_axis_name="subcore")


def _fixed_logistic_noise():
    u = jax.random.uniform(jax.random.key(42), (B, G, MC),
                           minval=1e-6, maxval=1.0 - 1e-6)
    lnoise = jnp.log(u) - jnp.log1p(-u)
    return np.asarray(lnoise.transpose(0, 2, 1).reshape(B * MC * G))


_LFLAT = _fixed_logistic_noise()

_CP = pltpu.CompilerParams(use_tc_tiling_on_sc=True)
if "needs_layout_passes" in pltpu.CompilerParams.__dataclass_fields__:
    _CP = dataclasses.replace(_CP, needs_layout_passes=False)


def _worker_id():
    return lax.axis_index("core") * NS + lax.axis_index("subcore")


def _sigmoid(v):
    return 1.0 / (1.0 + jnp.exp(-v))


BH = 56


def _k0_body(w_ref, x_ref, e_ref, pp_ref):
    x = x_ref[0]
    e = (x[0].astype(jnp.bfloat16).astype(jnp.float32) * w_ref[0]
         + x[1].astype(jnp.bfloat16).astype(jnp.float32) * w_ref[1]
         + x[2].astype(jnp.bfloat16).astype(jnp.float32) * w_ref[2]
         + w_ref[3])
    e_ref[0] = e
    pp_ref[0] = jax.nn.sigmoid(e)


NBANDS = HROWS // HB


def _k1_body(e_hbm, g_hbm, l_hbm, mask_hbm, gp_hbm,
             ev0, ev1, gv0, gv1, sums, counts, pvs, pvc,
             lv, gpv, table, sv0, sv1, shared,
             seme, semg, semm):
    wid = _worker_id()
    sid = lax.axis_index("subcore")
    b = wid // 2
    half = wid % 2
    hbase = half * HROWS
    evs, gvs = (ev0, ev1), (gv0, gv1)

    cl = pltpu.async_copy(l_hbm.at[pl.ds(b * MC * G, MC * G)], lv, semm)

    zero = jnp.zeros((NLANE,), jnp.float32)
    ones = jnp.full((NLANE,), 1.0, jnp.float32)

    @pl.loop(0, G, step=NLANE)
    def _(g):
        sums[pl.ds(g, NLANE)] = zero
        counts[pl.ds(g, NLANE)] = zero

    def fetch_a(k):
        h0 = hbase + k * HB
        ec = pltpu.async_copy(e_hbm.at[b, pl.ds(h0, HB)], evs[k % 2], seme)
        gc = pltpu.async_copy(g_hbm.at[b, pl.ds(h0, HB)], gvs[k % 2], semg)
        return ec, gc

    pend = fetch_a(0)
    for k in range(NBANDS):
        ec, gc = pend
        ec.wait()
        gc.wait()
        if k + 1 < NBANDS:
            pend = fetch_a(k + 1)
        ev, gv = evs[k % 2], gvs[k % 2]

        @pl.loop(0, HB)
        def _(r):
            for w in range(0, W, NLANE):
                sl = (r, pl.ds(w, NLANE))
                g = gv[sl]
                plsc.addupdate_scatter(sums, [g], ev[sl])
                plsc.addupdate_scatter(counts, [g], ones)

    pltpu.sync_copy(sums, shared.at[sid, 0])
    pltpu.sync_copy(counts, shared.at[sid, 1])
    plsc.subcore_barrier()
    pltpu.sync_copy(shared.at[sid ^ 1, 0], pvs)
    pltpu.sync_copy(shared.at[sid ^ 1, 1], pvc)
    cl.wait()

    iota8 = lax.iota(jnp.int32, NLANE) * MC

    @pl.loop(0, G, step=NLANE)
    def _(g):
        sl = pl.ds(g, NLANE)
        s = sums[sl] + pvs[sl]
        n = counts[sl] + pvc[sl]
        mu = s / jnp.maximum(n, 1.0)
        gpv[sl] = _sigmoid(mu)
        for m in range(MC):
            hard = jnp.where(mu + lv[pl.ds(m * G + g, NLANE)] > 0.0, 1.0, 0.0)
            plsc.store_scatter(table, [iota8 + (g * MC + m)], hard)

    @pl.when(half == 0)
    def _():
        pltpu.sync_copy(gpv, gp_hbm.at[pl.ds(b * G, G)])

    svs = (sv0, sv1)

    def fetch_b(k):
        h0 = hbase + k * HB
        return pltpu.async_copy(g_hbm.at[b, pl.ds(h0, HB)], gvs[k % 2], semg)

    pend = fetch_b(0)
    mcop = [None] * NBANDS
    for k in range(NBANDS):
        pend.wait()
        if k + 1 < NBANDS:
            pend = fetch_b(k + 1)
        if k >= 2:
            for h in mcop[k - 2]:
                h.wait()
        gv, selv = gvs[k % 2], svs[k % 2]

        @pl.loop(0, HB)
        def _(r):
            for w in range(0, W, NLANE):
                g8 = gv[r, pl.ds(w, NLANE)] * MC
                for m in range(MC):
                    selv[r, m, pl.ds(w, NLANE)] = plsc.load_gather(
                        table, [g8 + m])

        h0 = hbase + k * HB
        mcop[k] = [
            pltpu.async_copy(selv, mask_hbm.at[b, c, pl.ds(h0, HB)], semm)
            for c in range(C)
        ]

    for k in (NBANDS - 2, NBANDS - 1):
        for h in mcop[k]:
            h.wait()


def kernel(x, groups, W_pred, b_pred):
    w0bf = W_pred[0].astype(jnp.bfloat16).astype(jnp.float32)
    wvec = jnp.concatenate([w0bf, b_pred[0:1]])

    lflat = jnp.asarray(_LFLAT)

    e0, pp = pl.pallas_call(
        _k0_body,
        grid=(B, H // BH),
        in_specs=[
            pl.BlockSpec(memory_space=pltpu.SMEM),
            pl.BlockSpec((1, C, BH, W), lambda b, h: (b, 0, h, 0)),
        ],
        out_specs=[
            pl.BlockSpec((1, BH, W), lambda b, h: (b, h, 0)),
            pl.BlockSpec((1, BH, W), lambda b, h: (b, h, 0)),
        ],
        out_shape=[
            jax.ShapeDtypeStruct((B, H, W), jnp.float32),
            jax.ShapeDtypeStruct((B, H, W), jnp.float32),
        ],
    )(wvec, x)

    k1 = pl.kernel(
        _k1_body,
        out_type=[
            jax.ShapeDtypeStruct((B, C, H, MC, W), jnp.float32),
            jax.ShapeDtypeStruct((B * G,), jnp.float32),
        ],
        mesh=_MESH,
        compiler_params=_CP,
        scratch_types=[
            pltpu.VMEM((HB, W), jnp.float32),
            pltpu.VMEM((HB, W), jnp.float32),
            pltpu.VMEM((HB, W), jnp.int32),
            pltpu.VMEM((HB, W), jnp.int32),
            pltpu.VMEM((G,), jnp.float32),
            pltpu.VMEM((G,), jnp.float32),
            pltpu.VMEM((G,), jnp.float32),
            pltpu.VMEM((G,), jnp.float32),
            pltpu.VMEM((MC * G,), jnp.float32),
            pltpu.VMEM((G,), jnp.float32),
            pltpu.VMEM((G * MC,), jnp.float32),
            pltpu.VMEM((HB, MC, W), jnp.float32),
            pltpu.VMEM((HB, MC, W), jnp.float32),
            pltpu.VMEM_SHARED((NS, 2, G), jnp.float32),
            pltpu.SemaphoreType.DMA,
            pltpu.SemaphoreType.DMA,
            pltpu.SemaphoreType.DMA,
        ],
    )
    maskT, group_probs = k1(e0, groups, lflat)

    mask = maskT.transpose(0, 1, 2, 4, 3)
    return (mask, group_probs.reshape(B, G), pp)
